# Initial kernel scaffold; baseline (speedup 1.0000x reference)
import jax
import jax.numpy as jnp
from jax.experimental import pallas as pl


def kernel(betas, t_int):
    raise NotImplementedError("write your pallas kernel here")


# speedup vs baseline: 4.5621x; 4.5621x over previous
import functools

import jax
import jax.numpy as jnp
from jax import lax
from jax.experimental import pallas as pl
from jax.experimental.pallas import tpu as pltpu
from jax.experimental.pallas import tpu_sc as plsc

_LANES = 16
_NUM_CORES = 2
_NUM_SUBCORES = 16
_NUM_WORKERS = _NUM_CORES * _NUM_SUBCORES


@functools.lru_cache(maxsize=None)
def _build(num_t, num_idx):
    chunk = num_idx // _NUM_WORKERS
    mesh = plsc.VectorSubcoreMesh(core_axis_name="c", subcore_axis_name="s")

    @functools.partial(
        pl.kernel,
        out_type=jax.ShapeDtypeStruct((num_idx,), jnp.float32),
        mesh=mesh,
        scratch_types=[
            pltpu.VMEM((num_t,), jnp.float32),
            pltpu.VMEM((chunk,), jnp.int32),
            pltpu.VMEM((chunk,), jnp.float32),
        ],
        compiler_params=pltpu.CompilerParams(needs_layout_passes=False),
    )
    def lookup(betas_hbm, t_hbm, out_hbm, table_v, idx_v, out_v):
        wid = lax.axis_index("s") * _NUM_CORES + lax.axis_index("c")
        base = wid * chunk
        pltpu.sync_copy(betas_hbm, table_v)
        pltpu.sync_copy(t_hbm.at[pl.ds(base, chunk)], idx_v)
        for i in range(chunk // _LANES):
            idx = idx_v[pl.ds(i * _LANES, _LANES)]
            out_v[pl.ds(i * _LANES, _LANES)] = plsc.load_gather(table_v, [idx])
        pltpu.sync_copy(out_v, out_hbm.at[pl.ds(base, chunk)])

    return lookup


def kernel(betas, t_int):
    return _build(betas.shape[0], t_int.shape[0])(
        betas.astype(jnp.float32), t_int.astype(jnp.int32)
    )


# speedup vs baseline: 4.6520x; 1.0197x over previous
import functools

import jax
import jax.numpy as jnp
from jax import lax
from jax.experimental import pallas as pl
from jax.experimental.pallas import tpu as pltpu
from jax.experimental.pallas import tpu_sc as plsc

_LANES = 16
_NUM_CORES = 2
_NUM_SUBCORES = 16
_NUM_WORKERS = _NUM_CORES * _NUM_SUBCORES


@functools.lru_cache(maxsize=None)
def _build(num_t, num_idx):
    chunk = num_idx // _NUM_WORKERS
    mesh = plsc.VectorSubcoreMesh(core_axis_name="c", subcore_axis_name="s")

    @functools.partial(
        pl.kernel,
        out_type=jax.ShapeDtypeStruct((num_idx,), jnp.float32),
        mesh=mesh,
        scratch_types=[
            pltpu.VMEM((num_t,), jnp.float32),
            pltpu.VMEM((chunk,), jnp.int32),
            pltpu.VMEM((chunk,), jnp.float32),
            pltpu.SemaphoreType.DMA,
            pltpu.SemaphoreType.DMA,
        ],
        compiler_params=pltpu.CompilerParams(needs_layout_passes=False),
    )
    def lookup(betas_hbm, t_hbm, out_hbm, table_v, idx_v, out_v, sem_t, sem_i):
        wid = lax.axis_index("s") * _NUM_CORES + lax.axis_index("c")
        base = wid * chunk
        cp_t = pltpu.async_copy(betas_hbm, table_v, sem_t)
        cp_i = pltpu.async_copy(t_hbm.at[pl.ds(base, chunk)], idx_v, sem_i)
        cp_i.wait()
        cp_t.wait()
        for i in range(chunk // _LANES):
            idx = idx_v[pl.ds(i * _LANES, _LANES)]
            out_v[pl.ds(i * _LANES, _LANES)] = plsc.load_gather(table_v, [idx])
        pltpu.sync_copy(out_v, out_hbm.at[pl.ds(base, chunk)])

    return lookup


def kernel(betas, t_int):
    return _build(betas.shape[0], t_int.shape[0])(
        betas.astype(jnp.float32), t_int.astype(jnp.int32)
    )


# speedup vs baseline: 4.6715x; 1.0042x over previous
import functools

import jax
import jax.numpy as jnp
from jax import lax
from jax.experimental import pallas as pl
from jax.experimental.pallas import tpu as pltpu
from jax.experimental.pallas import tpu_sc as plsc

_LANES = 16
_NUM_CORES = 2
_NUM_SUBCORES = 16
_NUM_WORKERS = _NUM_CORES * _NUM_SUBCORES


@functools.lru_cache(maxsize=None)
def _build(num_t, num_idx):
    chunk = num_idx // _NUM_WORKERS
    mesh = plsc.VectorSubcoreMesh(core_axis_name="c", subcore_axis_name="s")

    @functools.partial(
        pl.kernel,
        out_type=jax.ShapeDtypeStruct((num_idx,), jnp.float32),
        mesh=mesh,
        scratch_types=[
            pltpu.VMEM((num_t,), jnp.float32),
            pltpu.VMEM((chunk,), jnp.int32),
            pltpu.VMEM((chunk,), jnp.float32),
            pltpu.SemaphoreType.DMA,
            pltpu.SemaphoreType.DMA,
        ],
        compiler_params=pltpu.CompilerParams(
            needs_layout_passes=False, skip_device_barrier=True
        ),
    )
    def lookup(betas_hbm, t_hbm, out_hbm, table_v, idx_v, out_v, sem_t, sem_i):
        wid = lax.axis_index("s") * _NUM_CORES + lax.axis_index("c")
        base = wid * chunk
        cp_t = pltpu.async_copy(betas_hbm, table_v, sem_t)
        cp_i = pltpu.async_copy(t_hbm.at[pl.ds(base, chunk)], idx_v, sem_i)
        cp_i.wait()
        cp_t.wait()
        for i in range(chunk // _LANES):
            idx = idx_v[pl.ds(i * _LANES, _LANES)]
            out_v[pl.ds(i * _LANES, _LANES)] = plsc.load_gather(table_v, [idx])
        pltpu.sync_copy(out_v, out_hbm.at[pl.ds(base, chunk)])

    return lookup


def kernel(betas, t_int):
    return _build(betas.shape[0], t_int.shape[0])(
        betas.astype(jnp.float32), t_int.astype(jnp.int32)
    )
